# baseline (speedup 1.0000x reference)
import functools

import jax
import jax.numpy as jnp
from jax import lax
from jax.experimental import pallas as pl
from jax.experimental.pallas import tpu as pltpu
from jax.experimental.pallas import tpu_sc as plsc

G = 128
H = 4
NEG = -1e30
EPS = 1e-7
TILE = 20000

NW = 32
K = 128
SP = 144


def _stats_body(x_ref, seg_ref, ws1_ref, ws2_ref, wt1_ref, wt2_ref,
                pg_ref, cnt_ref, m_ref, s_ref, n_ref):
    T, D = x_ref.shape
    HS = D // H
    i = pl.program_id(0)
    nt = pl.num_programs(0)

    @pl.when(i == 0)
    def _():
        m_ref[...] = jnp.full((G, 1), NEG, jnp.float32)
        s_ref[...] = jnp.zeros((G, H), jnp.float32)
        n_ref[...] = jnp.zeros((G, D), jnp.float32)
        cnt_ref[...] = jnp.zeros((G, 1), jnp.float32)

    x = x_ref[...]
    seg_row = seg_ref[0]
    mask = lax.broadcasted_iota(jnp.int32, (G, T), 0) == seg_row
    mask_f = mask.astype(jnp.float32)

    dn = (((1,), (0,)), ((), ()))
    h1 = jnp.maximum(
        lax.dot_general(x, ws1_ref[...], dn, preferred_element_type=jnp.float32), 0.0)
    sc = lax.dot_general(h1, ws2_ref[...], dn, preferred_element_type=jnp.float32)
    sc_t = lax.dot_general(ws2_ref[...], h1, (((0,), (1,)), ((), ())),
                           preferred_element_type=jnp.float32)
    v1 = jnp.maximum(
        lax.dot_general(x, wt1_ref[...], dn, preferred_element_type=jnp.float32), 0.0)
    vals = lax.dot_general(v1, wt2_ref[...], dn, preferred_element_type=jnp.float32)

    rowmax = jnp.max(sc_t, axis=0, keepdims=True)
    tm = jnp.max(jnp.where(mask, rowmax, NEG), axis=1, keepdims=True)
    m_old = m_ref[...]
    m_new = jnp.maximum(m_old, tm)
    m_ref[...] = m_new
    scale = jnp.exp(m_old - m_new)
    neg_m = jnp.where(m_new < NEG / 2, 0.0, jnp.exp(-m_new))

    esc = jnp.exp(sc)
    esc1 = jnp.concatenate([esc, jnp.ones((T, 1), jnp.float32)], axis=1)
    res = lax.dot_general(mask_f, esc1, (((1,), (0,)), ((), ())),
                          preferred_element_type=jnp.float32)
    s_ref[...] = s_ref[...] * scale + neg_m * res[:, :H]
    cnt_ref[...] = cnt_ref[...] + res[:, H:H + 1]

    rep = (lax.broadcasted_iota(jnp.int32, (H, D), 1) // HS
           == lax.broadcasted_iota(jnp.int32, (H, D), 0)).astype(jnp.float32)
    esc_exp = lax.dot_general(esc, rep, dn, preferred_element_type=jnp.float32)
    weighted = vals * esc_exp
    n_ref[...] = n_ref[...] * scale + neg_m * lax.dot_general(
        mask_f, weighted, (((1,), (0,)), ((), ())),
        preferred_element_type=jnp.float32)

    @pl.when(i == nt - 1)
    def _():
        s = s_ref[...]
        s_exp = jnp.concatenate(
            [jnp.broadcast_to(s[:, h:h + 1], (G, HS)) for h in range(H)], axis=1)
        pg_ref[...] = n_ref[...] / (s_exp + EPS)


def _per_graph_table(x, seg3d, ws1, ws2, wt1, wt2):
    V, D = x.shape
    nt = V // TILE
    return pl.pallas_call(
        _stats_body,
        grid=(nt,),
        in_specs=[
            pl.BlockSpec((TILE, D), lambda i: (i, 0)),
            pl.BlockSpec((1, 1, TILE), lambda i: (i, 0, 0)),
            pl.BlockSpec((D, D), lambda i: (0, 0)),
            pl.BlockSpec((D, H), lambda i: (0, 0)),
            pl.BlockSpec((D, D), lambda i: (0, 0)),
            pl.BlockSpec((D, D), lambda i: (0, 0)),
        ],
        out_specs=[
            pl.BlockSpec((G, D), lambda i: (0, 0)),
            pl.BlockSpec((G, 1), lambda i: (0, 0)),
        ],
        out_shape=[
            jax.ShapeDtypeStruct((G, D), jnp.float32),
            jax.ShapeDtypeStruct((G, 1), jnp.float32),
        ],
        scratch_shapes=[
            pltpu.VMEM((G, 1), jnp.float32),
            pltpu.VMEM((G, H), jnp.float32),
            pltpu.VMEM((G, D), jnp.float32),
        ],
    )(x, seg3d, ws1, ws2, wt1, wt2)


def _scatter_spans(table_flat, seg, starts_p, v, d):
    rpw = v // NW
    info = plsc.get_sparse_core_info()
    nc = info.num_cores
    mesh = plsc.VectorSubcoreMesh(core_axis_name="c", subcore_axis_name="s")

    @functools.partial(
        pl.kernel, mesh=mesh,
        out_type=jax.ShapeDtypeStruct((v * d,), jnp.float32),
        compiler_params=pltpu.CompilerParams(needs_layout_passes=False),
        scratch_types=[
            pltpu.VMEM((SP,), jnp.int32),
            pltpu.VMEM((16,), jnp.int32),
            pltpu.VMEM((16,), jnp.int32),
            pltpu.VMEM((d,), jnp.float32),
            pltpu.VMEM((K * d,), jnp.float32),
            pltpu.SemaphoreType.DMA,
        ],
    )
    def k(tbl_hbm, seg_hbm, st_hbm, out_hbm, st_v, sv_lo, sv_hi, trow, rbuf, wsem):
        wid = lax.axis_index("s") * nc + lax.axis_index("c")
        iota16 = lax.broadcasted_iota(jnp.int32, (16,), 0)
        lo = wid * rpw
        hi = lo + rpw
        pltpu.sync_copy(st_hbm, st_v)
        a_lo = (lo // 16) * 16
        a_hi = ((hi - 1) // 16) * 16
        pltpu.sync_copy(seg_hbm.at[pl.ds(a_lo, 16)], sv_lo)
        pltpu.sync_copy(seg_hbm.at[pl.ds(a_hi, 16)], sv_hi)

        def vext(vec, lane):
            picked = jnp.where(iota16 == lane, vec.astype(jnp.float32), 0.0)
            return jnp.sum(picked).astype(jnp.int32)

        g_lo = vext(sv_lo[...], lo - a_lo)
        g_hi = vext(sv_hi[...], hi - 1 - a_hi)

        svecs = [st_v[pl.ds(b * 16, 16)] for b in range(SP // 16)]

        def sread(idx):
            acc = jnp.float32(0)
            for b in range(SP // 16):
                acc = acc + jnp.sum(jnp.where(
                    iota16 + 16 * b == idx, svecs[b].astype(jnp.float32), 0.0))
            return acc.astype(jnp.int32)

        def drain(n):
            def dbody(_, c):
                pltpu.make_async_copy(
                    out_hbm.at[pl.ds(0, K * d)], rbuf, wsem).wait()
                return c
            lax.fori_loop(0, n, dbody, jnp.int32(0))

        def graph_body(g, prev_nfull):
            s_g = sread(g)
            e_g = sread(g + 1)
            s = jnp.maximum(s_g, lo)
            e = jnp.minimum(e_g, hi)
            span = e - s
            drain(prev_nfull)
            nfull = jnp.where(span > 0, span // K, 0).astype(jnp.int32)

            @pl.when(span > 0)
            def _():
                pltpu.sync_copy(tbl_hbm.at[pl.ds(g * d, d)], trow)
                regs = [trow[pl.ds(16 * r, 16)] for r in range(d // 16)]
                reps = jnp.minimum(span, K)

                def rep_body(kk, c):
                    for r in range(d // 16):
                        rbuf[pl.ds(kk * d + 16 * r, 16)] = regs[r]
                    return c
                lax.fori_loop(0, reps, rep_body, jnp.int32(0))

                def wr_body(i, c):
                    pltpu.async_copy(
                        rbuf.at[pl.ds(0, K * d)],
                        out_hbm.at[pl.ds((s + i * K) * d, K * d)], wsem)
                    return c
                lax.fori_loop(0, nfull, wr_body, jnp.int32(0))

                rem = span - nfull * K

                @pl.when((span >= K) & (rem > 0))
                def _():
                    pltpu.sync_copy(rbuf.at[pl.ds(0, K * d)],
                                    out_hbm.at[pl.ds((e - K) * d, K * d)])

                for sz in (64, 32, 16, 8, 4, 2, 1):
                    @pl.when((span >= sz) & (span < 2 * sz))
                    def _(sz=sz):
                        pltpu.sync_copy(rbuf.at[pl.ds(0, sz * d)],
                                        out_hbm.at[pl.ds(s * d, sz * d)])
                        pltpu.sync_copy(rbuf.at[pl.ds(0, sz * d)],
                                        out_hbm.at[pl.ds((e - sz) * d, sz * d)])

            return nfull

        last_nfull = lax.fori_loop(g_lo, g_hi + 1, graph_body, jnp.int32(0))
        drain(last_nfull)

    return k(table_flat, seg, starts_p)


def kernel(node_embeddings, node_to_graph_map, num_graphs, W_s1, W_s2, W_t1, W_t2):
    x = node_embeddings
    V, D = x.shape
    seg = node_to_graph_map.astype(jnp.int32)
    nt = V // TILE

    pg, cnt = _per_graph_table(x, seg.reshape(nt, 1, TILE), W_s1, W_s2, W_t1, W_t2)

    starts = jnp.cumsum(cnt[:, 0].astype(jnp.int32))
    starts_p = jnp.concatenate(
        [jnp.zeros((1,), jnp.int32), starts,
         jnp.full((SP - G - 1,), V, jnp.int32)])
    out_flat = _scatter_spans(pg.reshape(G * D), seg, starts_p, V, D)
    return out_flat.reshape(V, D)


# speedup vs baseline: 1.0052x; 1.0052x over previous
import functools

import jax
import jax.numpy as jnp
from jax import lax
from jax.experimental import pallas as pl
from jax.experimental.pallas import tpu as pltpu
from jax.experimental.pallas import tpu_sc as plsc

G = 128
H = 4
NEG = -1e30
EPS = 1e-7
TILE = 10000

NW = 32
K = 128
SP = 144


def _stats_body(x_ref, seg_ref, ws1_ref, ws2_ref, wt1_ref, wt2_ref,
                pg_ref, cnt_ref, m_ref, s_ref, n_ref):
    T, D = x_ref.shape
    HS = D // H
    i = pl.program_id(0)
    nt = pl.num_programs(0)

    @pl.when(i == 0)
    def _():
        m_ref[...] = jnp.full((G, 1), NEG, jnp.float32)
        s_ref[...] = jnp.zeros((G, H), jnp.float32)
        n_ref[...] = jnp.zeros((G, D), jnp.float32)
        cnt_ref[...] = jnp.zeros((G, 1), jnp.float32)

    x = x_ref[...]
    seg_row = seg_ref[0]
    mask = lax.broadcasted_iota(jnp.int32, (G, T), 0) == seg_row
    mask_f = mask.astype(jnp.float32)

    dn = (((1,), (0,)), ((), ()))
    h1 = jnp.maximum(
        lax.dot_general(x, ws1_ref[...], dn, preferred_element_type=jnp.float32), 0.0)
    sc = lax.dot_general(h1, ws2_ref[...], dn, preferred_element_type=jnp.float32)
    sc_t = lax.dot_general(ws2_ref[...], h1, (((0,), (1,)), ((), ())),
                           preferred_element_type=jnp.float32)
    v1 = jnp.maximum(
        lax.dot_general(x, wt1_ref[...], dn, preferred_element_type=jnp.float32), 0.0)
    vals = lax.dot_general(v1, wt2_ref[...], dn, preferred_element_type=jnp.float32)

    rowmax = jnp.max(sc_t, axis=0, keepdims=True)
    tm = jnp.max(jnp.where(mask, rowmax, NEG), axis=1, keepdims=True)
    m_old = m_ref[...]
    m_new = jnp.maximum(m_old, tm)
    m_ref[...] = m_new
    scale = jnp.exp(m_old - m_new)
    neg_m = jnp.where(m_new < NEG / 2, 0.0, jnp.exp(-m_new))

    esc = jnp.exp(sc)
    esc1 = jnp.concatenate([esc, jnp.ones((T, 1), jnp.float32)], axis=1)
    res = lax.dot_general(mask_f, esc1, (((1,), (0,)), ((), ())),
                          preferred_element_type=jnp.float32)
    s_ref[...] = s_ref[...] * scale + neg_m * res[:, :H]
    cnt_ref[...] = cnt_ref[...] + res[:, H:H + 1]

    rep = (lax.broadcasted_iota(jnp.int32, (H, D), 1) // HS
           == lax.broadcasted_iota(jnp.int32, (H, D), 0)).astype(jnp.float32)
    esc_exp = lax.dot_general(esc, rep, dn, preferred_element_type=jnp.float32)
    weighted = vals * esc_exp
    n_ref[...] = n_ref[...] * scale + neg_m * lax.dot_general(
        mask_f, weighted, (((1,), (0,)), ((), ())),
        preferred_element_type=jnp.float32)

    @pl.when(i == nt - 1)
    def _():
        s = s_ref[...]
        s_exp = jnp.concatenate(
            [jnp.broadcast_to(s[:, h:h + 1], (G, HS)) for h in range(H)], axis=1)
        pg_ref[...] = n_ref[...] / (s_exp + EPS)


def _per_graph_table(x, seg3d, ws1, ws2, wt1, wt2):
    V, D = x.shape
    nt = V // TILE
    return pl.pallas_call(
        _stats_body,
        grid=(nt,),
        in_specs=[
            pl.BlockSpec((TILE, D), lambda i: (i, 0)),
            pl.BlockSpec((1, 1, TILE), lambda i: (i, 0, 0)),
            pl.BlockSpec((D, D), lambda i: (0, 0)),
            pl.BlockSpec((D, H), lambda i: (0, 0)),
            pl.BlockSpec((D, D), lambda i: (0, 0)),
            pl.BlockSpec((D, D), lambda i: (0, 0)),
        ],
        out_specs=[
            pl.BlockSpec((G, D), lambda i: (0, 0)),
            pl.BlockSpec((G, 1), lambda i: (0, 0)),
        ],
        out_shape=[
            jax.ShapeDtypeStruct((G, D), jnp.float32),
            jax.ShapeDtypeStruct((G, 1), jnp.float32),
        ],
        scratch_shapes=[
            pltpu.VMEM((G, 1), jnp.float32),
            pltpu.VMEM((G, H), jnp.float32),
            pltpu.VMEM((G, D), jnp.float32),
        ],
    )(x, seg3d, ws1, ws2, wt1, wt2)


def _scatter_spans(table_flat, seg, starts_p, v, d):
    rpw = v // NW
    info = plsc.get_sparse_core_info()
    nc = info.num_cores
    mesh = plsc.VectorSubcoreMesh(core_axis_name="c", subcore_axis_name="s")

    @functools.partial(
        pl.kernel, mesh=mesh,
        out_type=jax.ShapeDtypeStruct((v * d,), jnp.float32),
        compiler_params=pltpu.CompilerParams(needs_layout_passes=False),
        scratch_types=[
            pltpu.VMEM((SP,), jnp.int32),
            pltpu.VMEM((16,), jnp.int32),
            pltpu.VMEM((16,), jnp.int32),
            pltpu.VMEM((d,), jnp.float32),
            pltpu.VMEM((K * d,), jnp.float32),
            pltpu.SemaphoreType.DMA,
        ],
    )
    def k(tbl_hbm, seg_hbm, st_hbm, out_hbm, st_v, sv_lo, sv_hi, trow, rbuf, wsem):
        wid = lax.axis_index("s") * nc + lax.axis_index("c")
        iota16 = lax.broadcasted_iota(jnp.int32, (16,), 0)
        lo = wid * rpw
        hi = lo + rpw
        pltpu.sync_copy(st_hbm, st_v)
        a_lo = (lo // 16) * 16
        a_hi = ((hi - 1) // 16) * 16
        pltpu.sync_copy(seg_hbm.at[pl.ds(a_lo, 16)], sv_lo)
        pltpu.sync_copy(seg_hbm.at[pl.ds(a_hi, 16)], sv_hi)

        def vext(vec, lane):
            picked = jnp.where(iota16 == lane, vec.astype(jnp.float32), 0.0)
            return jnp.sum(picked).astype(jnp.int32)

        g_lo = vext(sv_lo[...], lo - a_lo)
        g_hi = vext(sv_hi[...], hi - 1 - a_hi)

        svecs = [st_v[pl.ds(b * 16, 16)] for b in range(SP // 16)]

        def sread(idx):
            acc = jnp.float32(0)
            for b in range(SP // 16):
                acc = acc + jnp.sum(jnp.where(
                    iota16 + 16 * b == idx, svecs[b].astype(jnp.float32), 0.0))
            return acc.astype(jnp.int32)

        def drain(n):
            def dbody(_, c):
                pltpu.make_async_copy(
                    out_hbm.at[pl.ds(0, K * d)], rbuf, wsem).wait()
                return c
            lax.fori_loop(0, n, dbody, jnp.int32(0))

        def graph_body(g, prev_nfull):
            s_g = sread(g)
            e_g = sread(g + 1)
            s = jnp.maximum(s_g, lo)
            e = jnp.minimum(e_g, hi)
            span = e - s
            drain(prev_nfull)
            nfull = jnp.where(span > 0, span // K, 0).astype(jnp.int32)

            @pl.when(span > 0)
            def _():
                pltpu.sync_copy(tbl_hbm.at[pl.ds(g * d, d)], trow)
                regs = [trow[pl.ds(16 * r, 16)] for r in range(d // 16)]
                reps = jnp.minimum(span, K)

                def rep_body(kk, c):
                    for r in range(d // 16):
                        rbuf[pl.ds(kk * d + 16 * r, 16)] = regs[r]
                    return c
                lax.fori_loop(0, reps, rep_body, jnp.int32(0))

                def wr_body(i, c):
                    pltpu.async_copy(
                        rbuf.at[pl.ds(0, K * d)],
                        out_hbm.at[pl.ds((s + i * K) * d, K * d)], wsem)
                    return c
                lax.fori_loop(0, nfull, wr_body, jnp.int32(0))

                rem = span - nfull * K

                @pl.when((span >= K) & (rem > 0))
                def _():
                    pltpu.sync_copy(rbuf.at[pl.ds(0, K * d)],
                                    out_hbm.at[pl.ds((e - K) * d, K * d)])

                for sz in (64, 32, 16, 8, 4, 2, 1):
                    @pl.when((span >= sz) & (span < 2 * sz))
                    def _(sz=sz):
                        pltpu.sync_copy(rbuf.at[pl.ds(0, sz * d)],
                                        out_hbm.at[pl.ds(s * d, sz * d)])
                        pltpu.sync_copy(rbuf.at[pl.ds(0, sz * d)],
                                        out_hbm.at[pl.ds((e - sz) * d, sz * d)])

            return nfull

        last_nfull = lax.fori_loop(g_lo, g_hi + 1, graph_body, jnp.int32(0))
        drain(last_nfull)

    return k(table_flat, seg, starts_p)


def kernel(node_embeddings, node_to_graph_map, num_graphs, W_s1, W_s2, W_t1, W_t2):
    x = node_embeddings
    V, D = x.shape
    seg = node_to_graph_map.astype(jnp.int32)
    nt = V // TILE

    pg, cnt = _per_graph_table(x, seg.reshape(nt, 1, TILE), W_s1, W_s2, W_t1, W_t2)

    starts = jnp.cumsum(cnt[:, 0].astype(jnp.int32))
    starts_p = jnp.concatenate(
        [jnp.zeros((1,), jnp.int32), starts,
         jnp.full((SP - G - 1,), V, jnp.int32)])
    out_flat = _scatter_spans(pg.reshape(G * D), seg, starts_p, V, D)
    return out_flat.reshape(V, D)
